# baseline (speedup 1.0000x reference)
import functools

import jax
import jax.numpy as jnp
from jax import lax
from jax.experimental import pallas as pl
from jax.experimental.pallas import tpu as pltpu
from jax.experimental.pallas import tpu_sc as plsc

B_POS = 16384
B_NEG = 81920
B_TOT = B_POS + B_NEG
R_TAB = 1999999
D = 64
L = 16
IDX_W = 128
PK = 128
R_PK = (R_TAB - 1) // 2
LAST = R_TAB - 1

NC = 2
NS = 16
NW = NC * NS

ROWS_W = B_TOT // NW
CH = 256
N_CH = ROWS_W // CH
G_CH = CH // L


def _sc_scores(u_idx, v_idx, Upk, Vpk, lastU, lastV):
  mesh = plsc.VectorSubcoreMesh(core_axis_name="c", subcore_axis_name="s")

  @functools.partial(
      pl.kernel,
      out_type=jax.ShapeDtypeStruct((B_TOT,), jnp.float32),
      mesh=mesh,
      scratch_types=[
          pltpu.VMEM((ROWS_W,), jnp.int32),
          pltpu.VMEM((ROWS_W,), jnp.int32),
          pltpu.VMEM((ROWS_W,), jnp.int32),
          pltpu.VMEM((ROWS_W,), jnp.int32),
          pltpu.VMEM((CH, PK), jnp.float32),
          pltpu.VMEM((CH, PK), jnp.float32),
          pltpu.VMEM((D,), jnp.float32),
          pltpu.VMEM((D,), jnp.float32),
          pltpu.VMEM((ROWS_W,), jnp.float32),
          pltpu.SemaphoreType.DMA,
      ],
  )
  def k(u_idx_hbm, v_idx_hbm, u_hbm, v_hbm, lu_hbm, lv_hbm, out_hbm,
        uix, vix, urix, vrix, urows, vrows, lu, lv, sc, sem):
    wid = lax.axis_index("s") * NC + lax.axis_index("c")
    base = wid * ROWS_W
    pltpu.sync_copy(u_idx_hbm.at[pl.ds(base, ROWS_W)], uix)
    pltpu.sync_copy(v_idx_hbm.at[pl.ds(base, ROWS_W)], vix)
    pltpu.sync_copy(lu_hbm, lu)
    pltpu.sync_copy(lv_hbm, lv)

    @plsc.parallel_loop(0, ROWS_W // L, unroll=4)
    def _(t):
      s = t * L
      urix[pl.ds(s, L)] = jnp.minimum(
          lax.shift_right_logical(uix[pl.ds(s, L)], 1), R_PK - 1)
      vrix[pl.ds(s, L)] = jnp.minimum(
          lax.shift_right_logical(vix[pl.ds(s, L)], 1), R_PK - 1)

    lane = lax.iota(jnp.int32, L)
    perms = [lane ^ dd for dd in (8, 4, 2, 1)]
    lu_h = [lu[pl.ds(h * L, L)] for h in range(D // L)]
    lv_h = [lv[pl.ds(h * L, L)] for h in range(D // L)]

    def chunk_body(c, _):
      cb = c * CH
      dmas = []
      for j in range(CH // IDX_W):
        dmas.append(pltpu.async_copy(
            u_hbm.at[urix.at[pl.ds(cb + j * IDX_W, IDX_W)]],
            urows.at[pl.ds(j * IDX_W, IDX_W)], sem))
        dmas.append(pltpu.async_copy(
            v_hbm.at[vrix.at[pl.ds(cb + j * IDX_W, IDX_W)]],
            vrows.at[pl.ds(j * IDX_W, IDX_W)], sem))
      for dma in dmas:
        dma.wait()

      @plsc.parallel_loop(0, G_CH)
      def _(g):
        gb = g * L
        uiv = uix[pl.ds(cb + gb, L)]
        viv = vix[pl.ds(cb + gb, L)]
        svec = jnp.zeros((L,), jnp.float32)
        for l in range(L):
          ru = uiv[l]
          rv = viv[l]
          offu = (ru & 1) * D
          offv = (rv & 1) * D
          lau = ru >= LAST
          lav = rv >= LAST
          q = gb + l
          acc = jnp.zeros((L,), jnp.float32)
          for h in range(D // L):
            xu = urows[q, pl.ds(offu + h * L, L)]
            xv = vrows[q, pl.ds(offv + h * L, L)]
            xu = jnp.where(lau, lu_h[h], xu)
            xv = jnp.where(lav, lv_h[h], xv)
            acc = acc + xu * xv
          for perm in perms:
            acc = acc + acc.at[perm].get(mode="promise_in_bounds",
                                         unique_indices=True)
          svec = jnp.where(lane == l, acc, svec)
        sc[pl.ds(cb + gb, L)] = svec

    lax.fori_loop(0, N_CH, chunk_body, None)
    pltpu.sync_copy(sc, out_hbm.at[pl.ds(base, ROWS_W)])

  return k(u_idx, v_idx, Upk, Vpk, lastU, lastV)


def _tc_loss(scores):
  x = scores.reshape(B_TOT // 128, 128)
  pos_rows = B_POS // 128

  def body(x_ref, o_ref):
    xv = x_ref[...]
    row = lax.broadcasted_iota(jnp.int32, xv.shape, 0)
    sgn = jnp.where(row < pos_rows, 1.0, -1.0)
    o_ref[0, 0] = -jnp.sum(jax.nn.log_sigmoid(xv * sgn))

  out = pl.pallas_call(
      body,
      out_shape=jax.ShapeDtypeStruct((1, 1), jnp.float32),
      out_specs=pl.BlockSpec(memory_space=pltpu.SMEM),
  )(x)
  return out[0, 0]


@jax.jit
def kernel(pos_u, pos_v, neg_u, neg_v, U, V):
  u_idx = jnp.concatenate([pos_u, neg_u]).astype(jnp.int32)
  v_idx = jnp.concatenate([pos_v, neg_v]).astype(jnp.int32)
  Upk = U[:R_TAB - 1].reshape(R_PK, PK)
  Vpk = V[:R_TAB - 1].reshape(R_PK, PK)
  scores = _sc_scores(u_idx, v_idx, Upk, Vpk, U[R_TAB - 1], V[R_TAB - 1])
  return _tc_loss(scores)
